# baseline (speedup 1.0000x reference)
import functools

import jax
import jax.numpy as jnp
from jax import lax
from jax.experimental import pallas as pl
from jax.experimental.pallas import tpu as pltpu
from jax.experimental.pallas import tpu_sc as plsc

NC = 2
NS = 16
NW = NC * NS
LANES = 16


def _chunks(n):
  out = []
  off = 0
  while off < n:
    cnt = min(128, n - off)
    out.append((off, cnt))
    off += cnt
  return out


@functools.lru_cache(maxsize=None)
def _build_sc_gather_mean(B, L, Lp, D, group_sents):
  sent_per_w = B // NW
  group_rows = group_sents * L
  n_groups = sent_per_w // group_sents
  assert n_groups % 2 == 0 and Lp % 8 == 0 and L <= 128
  nc = D // LANES
  inv_l = 1.0 / L

  def body(idx_hbm, table_hbm, ave_hbm, idx_v, rows_v, ave_v, sem0, sem1):
    sems = (sem0, sem1)
    wid = lax.axis_index("s") * NC + lax.axis_index("c")
    pltpu.sync_copy(idx_hbm.at[pl.ds(wid * sent_per_w, sent_per_w)], idx_v)

    def issue(g, buf):
      for s2 in range(group_sents):
        idx_row = idx_v.at[g * group_sents + s2]
        pltpu.async_copy(
            table_hbm.at[idx_row.at[pl.ds(0, L)]],
            rows_v.at[buf].at[pl.ds(s2 * L, L)],
            sems[buf])

    def drain(buf):
      pltpu.make_async_copy(
          table_hbm.at[pl.ds(0, group_rows)], rows_v.at[buf], sems[buf]).wait()

    def reduce(g, buf):
      def sent(s, carry):
        accs = [jnp.zeros((LANES,), jnp.float32)] * nc

        def red(j, accs):
          return tuple(
              accs[c] + rows_v[buf, s * L + j, pl.ds(c * LANES, LANES)]
              for c in range(nc))

        accs = lax.fori_loop(0, L, red, tuple(accs), unroll=5)
        for c in range(nc):
          ave_v[g * group_sents + s, pl.ds(c * LANES, LANES)] = (
              accs[c] * inv_l)
        return carry

      lax.fori_loop(0, group_sents, sent, 0)

    issue(0, 0)

    def body2(i, carry):
      g0 = 2 * i
      issue(g0 + 1, 1)
      drain(0)
      reduce(g0, 0)

      @pl.when(g0 + 2 < n_groups)
      def _():
        issue(g0 + 2, 0)

      drain(1)
      reduce(g0 + 1, 1)
      return carry

    lax.fori_loop(0, n_groups // 2, body2, 0)
    pltpu.sync_copy(ave_v, ave_hbm.at[pl.ds(wid * sent_per_w, sent_per_w)])

  return pl.kernel(
      body,
      out_type=jax.ShapeDtypeStruct((B, D), jnp.float32),
      mesh=plsc.VectorSubcoreMesh(core_axis_name="c", subcore_axis_name="s",
                                  num_cores=NC, num_subcores=NS),
      compiler_params=pltpu.CompilerParams(use_tc_tiling_on_sc=False),
      scratch_types=[
          pltpu.VMEM((sent_per_w, Lp), jnp.int32),
          pltpu.VMEM((2, group_rows, D), jnp.float32),
          pltpu.VMEM((sent_per_w, D), jnp.float32),
          pltpu.SemaphoreType.DMA,
          pltpu.SemaphoreType.DMA,
      ],
  )


def _tc_head_body(ave_ref, wm_ref, wclf_ref, b_ref, out_ref):
  wc = jnp.dot(wclf_ref[...], wm_ref[...], preferred_element_type=jnp.float32)
  logits = lax.dot_general(ave_ref[...], wc, (((1,), (1,)), ((), ())),
                           preferred_element_type=jnp.float32)
  logits = logits + b_ref[...]
  m = jnp.max(logits, axis=-1, keepdims=True)
  e = jnp.exp(logits - m)
  out_ref[...] = e / jnp.sum(e, axis=-1, keepdims=True)


@functools.lru_cache(maxsize=None)
def _build_tc_head(B, OUT):
  return pl.pallas_call(
      _tc_head_body,
      out_shape=jax.ShapeDtypeStruct((B, OUT), jnp.float32),
  )


def kernel(indices, table, W_m, W_clf, b_clf):
  B, L = indices.shape
  V, D = table.shape
  OUT = W_clf.shape[0]
  group_sents = 8
  Lp = -(-L // 8) * 8
  assert B % NW == 0 and D % LANES == 0
  sc_gather_mean = _build_sc_gather_mean(B, L, Lp, D, group_sents)
  tc_head = _build_tc_head(B, OUT)
  idx_p = indices if Lp == L else jnp.pad(indices, ((0, 0), (0, Lp - L)))
  ave = sc_gather_mean(idx_p, table)
  return tc_head(ave, W_m, W_clf, b_clf.reshape(1, OUT))


# speedup vs baseline: 1.0679x; 1.0679x over previous
import functools

import jax
import jax.numpy as jnp
from jax import lax
from jax.experimental import pallas as pl
from jax.experimental.pallas import tpu as pltpu
from jax.experimental.pallas import tpu_sc as plsc

NC = 2
NS = 16
NW = NC * NS
LANES = 16


def _chunks(n):
  out = []
  off = 0
  while off < n:
    cnt = min(128, n - off)
    out.append((off, cnt))
    off += cnt
  return out


_NST = 8


def _tc_proj_body(rb, OUT, n_blocks, t_hbm, wm_ref, wclf_ref, out_ref,
                  tb, sems):
  i = pl.program_id(0)
  sub = rb // 8
  st = rb // _NST

  def issue(c, b):
    for s in range(_NST):
      pltpu.make_async_copy(
          t_hbm.at[pl.ds(c * rb + s * st, st), :],
          tb.at[b].at[pl.ds(s * st, st)],
          sems.at[b, s]).start()

  def drain(b):
    for s in range(_NST):
      pltpu.make_async_copy(
          t_hbm.at[pl.ds(s * st, st), :],
          tb.at[b].at[pl.ds(s * st, st)],
          sems.at[b, s]).wait()

  @pl.when(i == 0)
  def _():
    issue(0, 0)

  @pl.when(i + 1 < n_blocks)
  def _():
    issue(i + 1, (i + 1) % 2)

  b = i % 2
  drain(b)
  wc = jnp.dot(wclf_ref[...], wm_ref[...],
               preferred_element_type=jnp.float32)
  wc16 = jnp.concatenate(
      [wc, jnp.zeros((LANES - OUT, wc.shape[1]), jnp.float32)], axis=0)
  for a in range(8):
    p = lax.dot_general(tb[b, pl.ds(a * sub, sub), :], wc16,
                        (((1,), (1,)), ((), ())),
                        preferred_element_type=jnp.float32)
    out_ref[:, pl.ds(a * LANES, LANES)] = p


@functools.lru_cache(maxsize=None)
def _build_tc_proj(V, D, OUT, rb):
  n_blocks = V // rb
  return pl.pallas_call(
      functools.partial(_tc_proj_body, rb, OUT, n_blocks),
      grid=(n_blocks,),
      in_specs=[
          pl.BlockSpec(memory_space=pl.ANY),
          pl.BlockSpec((D, D), lambda i: (0, 0)),
          pl.BlockSpec((OUT, D), lambda i: (0, 0)),
      ],
      out_specs=pl.BlockSpec((rb // 8, 128), lambda i: (i, 0)),
      out_shape=jax.ShapeDtypeStruct((V * LANES // 128, 128), jnp.float32),
      scratch_shapes=[
          pltpu.VMEM((2, rb, D), jnp.float32),
          pltpu.SemaphoreType.DMA((2, _NST)),
      ],
  )


@functools.lru_cache(maxsize=None)
def _build_sc_gather_sum(B, L, V, group_sents):
  sent_per_w = B // NW
  group_rows = group_sents * L
  n_groups = sent_per_w // group_sents
  assert n_groups % 2 == 0 and group_rows % 8 == 0
  chunks = _chunks(group_rows)

  def body(idx_hbm, p_hbm, out_hbm, idx_v, rows_v, out_v, sem0, sem1):
    sems = (sem0, sem1)
    wid = lax.axis_index("s") * NC + lax.axis_index("c")
    pltpu.sync_copy(idx_hbm.at[pl.ds(wid * n_groups, n_groups)], idx_v)

    def issue(g, buf):
      idx_row = idx_v.at[g]
      for off, cnt in chunks:
        pltpu.async_copy(
            p_hbm.at[idx_row.at[pl.ds(off, cnt)]],
            rows_v.at[buf].at[pl.ds(off, cnt)],
            sems[buf])

    def drain(buf):
      pltpu.make_async_copy(
          p_hbm.at[pl.ds(0, group_rows)], rows_v.at[buf], sems[buf]).wait()

    def reduce(g, buf):
      for s in range(group_sents):
        def red(j, acc, s=s, buf=buf):
          return acc + rows_v[buf, s * L + j, :]
        acc = lax.fori_loop(0, L, red, jnp.zeros((LANES,), jnp.float32),
                            unroll=10)
        out_v[g * group_sents + s, :] = acc

    issue(0, 0)

    def body2(i, carry):
      g0 = 2 * i
      issue(g0 + 1, 1)
      drain(0)
      reduce(g0, 0)

      @pl.when(g0 + 2 < n_groups)
      def _():
        issue(g0 + 2, 0)

      drain(1)
      reduce(g0 + 1, 1)
      return carry

    lax.fori_loop(0, n_groups // 2, body2, 0)
    pltpu.sync_copy(out_v, out_hbm.at[pl.ds(wid * sent_per_w, sent_per_w)])

  return pl.kernel(
      body,
      out_type=jax.ShapeDtypeStruct((B, LANES), jnp.float32),
      mesh=plsc.VectorSubcoreMesh(core_axis_name="c", subcore_axis_name="s",
                                  num_cores=NC, num_subcores=NS),
      compiler_params=pltpu.CompilerParams(use_tc_tiling_on_sc=False),
      scratch_types=[
          pltpu.VMEM((n_groups, group_rows), jnp.int32),
          pltpu.VMEM((2, group_rows, LANES), jnp.float32),
          pltpu.VMEM((sent_per_w, LANES), jnp.float32),
          pltpu.SemaphoreType.DMA,
          pltpu.SemaphoreType.DMA,
      ],
  )


def _tc_head_body(L, OUT, sums_ref, b_ref, out_ref):
  logits = sums_ref[...][:, :OUT] * (1.0 / L) + b_ref[...]
  m = jnp.max(logits, axis=-1, keepdims=True)
  e = jnp.exp(logits - m)
  out_ref[...] = e / jnp.sum(e, axis=-1, keepdims=True)


@functools.lru_cache(maxsize=None)
def _build_tc_head(B, L, OUT):
  return pl.pallas_call(
      functools.partial(_tc_head_body, L, OUT),
      out_shape=jax.ShapeDtypeStruct((B, OUT), jnp.float32),
  )


def kernel(indices, table, W_m, W_clf, b_clf):
  B, L = indices.shape
  V, D = table.shape
  OUT = W_clf.shape[0]
  group_sents = 32
  rb = 40000
  sub = rb // 8
  assert B % NW == 0 and OUT <= LANES and V % rb == 0
  tc_proj = _build_tc_proj(V, D, OUT, rb)
  sc_gather_sum = _build_sc_gather_sum(B, L, V, group_sents)
  tc_head = _build_tc_head(B, L, OUT)

  p = tc_proj(table, W_m, W_clf)
  p16 = jnp.reshape(p, (V, LANES))
  w = indices % rb
  ridx = (indices - w) + (w % sub) * 8 + w // sub
  idx3 = ridx.reshape(B // group_sents, group_sents * L)
  sums = sc_gather_sum(idx3, p16)
  return tc_head(sums, b_clf.reshape(1, OUT))


# speedup vs baseline: 1.1165x; 1.0456x over previous
import functools

import jax
import jax.numpy as jnp
from jax import lax
from jax.experimental import pallas as pl
from jax.experimental.pallas import tpu as pltpu
from jax.experimental.pallas import tpu_sc as plsc

NC = 2
NS = 16
NW = NC * NS
LANES = 16


def _chunks(n):
  out = []
  off = 0
  while off < n:
    cnt = min(128, n - off)
    out.append((off, cnt))
    off += cnt
  return out


_NST = 8


def _tc_proj_body(rb, OUT, n_blocks, t_hbm, wm_ref, wclf_ref, out_ref,
                  tb, sems):
  i = pl.program_id(0)
  sub = rb // 8
  st = rb // _NST

  def issue(c, b):
    for s in range(_NST):
      pltpu.make_async_copy(
          t_hbm.at[pl.ds(c * rb + s * st, st), :],
          tb.at[b].at[pl.ds(s * st, st)],
          sems.at[b, s]).start()

  def drain(b):
    for s in range(_NST):
      pltpu.make_async_copy(
          t_hbm.at[pl.ds(s * st, st), :],
          tb.at[b].at[pl.ds(s * st, st)],
          sems.at[b, s]).wait()

  @pl.when(i == 0)
  def _():
    issue(0, 0)

  @pl.when(i + 1 < n_blocks)
  def _():
    issue(i + 1, (i + 1) % 2)

  b = i % 2
  drain(b)
  wc = jnp.dot(wclf_ref[...], wm_ref[...],
               preferred_element_type=jnp.float32)
  wc16 = jnp.concatenate(
      [wc, jnp.zeros((LANES - OUT, wc.shape[1]), jnp.float32)], axis=0)
  for a in range(8):
    p = lax.dot_general(tb[b, pl.ds(a * sub, sub), :], wc16,
                        (((1,), (1,)), ((), ())),
                        preferred_element_type=jnp.float32)
    out_ref[:, pl.ds(a * LANES, LANES)] = p


@functools.lru_cache(maxsize=None)
def _build_tc_proj(V, D, OUT, rb):
  n_blocks = V // rb
  return pl.pallas_call(
      functools.partial(_tc_proj_body, rb, OUT, n_blocks),
      grid=(n_blocks,),
      in_specs=[
          pl.BlockSpec(memory_space=pl.ANY),
          pl.BlockSpec((D, D), lambda i: (0, 0)),
          pl.BlockSpec((OUT, D), lambda i: (0, 0)),
      ],
      out_specs=pl.BlockSpec((rb // 8, 128), lambda i: (i, 0)),
      out_shape=jax.ShapeDtypeStruct((V * LANES // 128, 128), jnp.float32),
      scratch_shapes=[
          pltpu.VMEM((2, rb, D), jnp.float32),
          pltpu.SemaphoreType.DMA((2, _NST)),
      ],
  )


@functools.lru_cache(maxsize=None)
def _build_sc_gather_sum(B, L, V, group_sents):
  sent_per_w = B // NW
  group_rows = group_sents * L
  n_groups = sent_per_w // group_sents
  assert n_groups % 2 == 0 and group_rows % 8 == 0
  chunks = _chunks(group_rows)

  def body(idx_hbm, p_hbm, out_hbm, idx_v, rows_v, out_v, sem0, sem1):
    sems = (sem0, sem1)
    wid = lax.axis_index("s") * NC + lax.axis_index("c")
    pltpu.sync_copy(idx_hbm.at[pl.ds(wid * n_groups, n_groups)], idx_v)

    def issue(g, buf):
      idx_row = idx_v.at[g]
      for off, cnt in chunks:
        pltpu.async_copy(
            p_hbm.at[idx_row.at[pl.ds(off, cnt)]],
            rows_v.at[buf].at[pl.ds(off, cnt)],
            sems[buf])

    def drain(buf):
      pltpu.make_async_copy(
          p_hbm.at[pl.ds(0, group_rows)], rows_v.at[buf], sems[buf]).wait()

    def reduce(g, buf):
      for s in range(group_sents):
        def red(j, acc, s=s, buf=buf):
          return acc + rows_v[buf, s * L + j, :]
        acc = lax.fori_loop(0, L, red, jnp.zeros((LANES,), jnp.float32),
                            unroll=10)
        out_v[g * group_sents + s, :] = acc

    issue(0, 0)

    def body2(i, carry):
      g0 = 2 * i
      issue(g0 + 1, 1)
      drain(0)
      reduce(g0, 0)

      @pl.when(g0 + 2 < n_groups)
      def _():
        issue(g0 + 2, 0)

      drain(1)
      reduce(g0 + 1, 1)
      return carry

    lax.fori_loop(0, n_groups // 2, body2, 0)
    pltpu.sync_copy(out_v, out_hbm.at[pl.ds(wid * sent_per_w, sent_per_w)])

  return pl.kernel(
      body,
      out_type=jax.ShapeDtypeStruct((B, LANES), jnp.float32),
      mesh=plsc.VectorSubcoreMesh(core_axis_name="c", subcore_axis_name="s",
                                  num_cores=NC, num_subcores=NS),
      compiler_params=pltpu.CompilerParams(use_tc_tiling_on_sc=False),
      scratch_types=[
          pltpu.VMEM((n_groups, group_rows), jnp.int32),
          pltpu.VMEM((2, group_rows, LANES), jnp.float32),
          pltpu.VMEM((sent_per_w, LANES), jnp.float32),
          pltpu.SemaphoreType.DMA,
          pltpu.SemaphoreType.DMA,
      ],
  )


def _tc_head_body(L, OUT, sums_ref, b_ref, out_ref):
  logits = sums_ref[...][:, :OUT] * (1.0 / L) + b_ref[...]
  m = jnp.max(logits, axis=-1, keepdims=True)
  e = jnp.exp(logits - m)
  out_ref[...] = e / jnp.sum(e, axis=-1, keepdims=True)


@functools.lru_cache(maxsize=None)
def _build_tc_head(B, L, OUT):
  return pl.pallas_call(
      functools.partial(_tc_head_body, L, OUT),
      out_shape=jax.ShapeDtypeStruct((B, OUT), jnp.float32),
  )


def kernel(indices, table, W_m, W_clf, b_clf):
  B, L = indices.shape
  V, D = table.shape
  OUT = W_clf.shape[0]
  group_sents = 8
  rb = 40000
  sub = rb // 8
  assert B % NW == 0 and OUT <= LANES and V % rb == 0
  tc_proj = _build_tc_proj(V, D, OUT, rb)
  sc_gather_sum = _build_sc_gather_sum(B, L, V, group_sents)
  tc_head = _build_tc_head(B, L, OUT)

  p = tc_proj(table, W_m, W_clf)
  p16 = jnp.reshape(p, (V, LANES))
  w = indices % rb
  ridx = (indices - w) + (w % sub) * 8 + w // sub
  idx3 = ridx.reshape(B // group_sents, group_sents * L)
  sums = sc_gather_sum(idx3, p16)
  return tc_head(sums, b_clf.reshape(1, OUT))
